# baseline (speedup 1.0000x reference)
import functools

import jax
import jax.numpy as jnp
from jax import lax
from jax.experimental import pallas as pl
from jax.experimental.pallas import tpu as pltpu
from jax.experimental.pallas import tpu_sc as plsc

VOCAB = 49408
EMBED = 512
NUM_POS = 77
BATCH = 1024
SEQ = 77
TOTAL = BATCH * SEQ
NUM_WORKERS = 32
ROWS_PER_WORKER = TOTAL // NUM_WORKERS
CHUNK = 56
NCHUNKS = ROWS_PER_WORKER // CHUNK
NBUF = 3
LANES = 16
VREGS_PER_ROW = EMBED // LANES


def _emb_body(ids_hbm, table_hbm, pos_hbm, out_hbm, idx_v, b0, b1, b2, pos_v,
              g0, g1, g2, w0, w1, w2):
  bufs = (b0, b1, b2)
  gsems = (g0, g1, g2)
  wsems = (w0, w1, w2)
  num_cores = 2
  wid = lax.axis_index("s") * num_cores + lax.axis_index("c")
  base = wid * ROWS_PER_WORKER

  pltpu.sync_copy(ids_hbm.at[pl.ds(base, ROWS_PER_WORKER)], idx_v)
  pltpu.sync_copy(pos_hbm, pos_v)

  def gather_start(c, b):
    pltpu.async_copy(
        table_hbm.at[idx_v.at[pl.ds(c * CHUNK, CHUNK)]], bufs[b], gsems[b]
    )

  def gather_wait(b):
    pltpu.make_async_copy(
        table_hbm.at[idx_v.at[pl.ds(0, CHUNK)]], bufs[b], gsems[b]
    ).wait()

  def write_start(c, b):
    pltpu.async_copy(bufs[b], out_hbm.at[pl.ds(base + c * CHUNK, CHUNK)],
                     wsems[b])

  def write_wait(b):
    pltpu.make_async_copy(
        bufs[b], out_hbm.at[pl.ds(0, CHUNK)], wsems[b]
    ).wait()

  def add_pos(c, buf):
    s0 = lax.rem(c * CHUNK, NUM_POS)

    def body(k, cc):
      rk = s0 + k
      r = lax.select(rk >= NUM_POS, rk - NUM_POS, rk)
      for j in range(VREGS_PER_ROW):
        sl = pl.ds(j * LANES, LANES)
        buf[k, sl] = buf[k, sl] + pos_v[r, sl]
      return cc

    lax.fori_loop(0, CHUNK, body, 0, unroll=2)

  def step(c, b, first):
    gather_wait(b)
    if not first:
      write_wait((b + 2) % NBUF)
    gather_start(c + 2, (b + 2) % NBUF)
    add_pos(c, bufs[b])
    write_start(c, b)

  gather_start(0, 0)
  gather_start(1, 1)
  step(0, 0, True)
  step(1, 1, False)
  step(2, 2, False)

  def triple(i, carry):
    c0 = 3 * i
    for b in range(NBUF):
      step(c0 + b, b, False)
    return carry

  lax.fori_loop(1, NCHUNKS // NBUF, triple, 0)

  for c in (NCHUNKS - 2, NCHUNKS - 1):
    b = c % NBUF
    gather_wait(b)
    add_pos(c, bufs[b])
    write_start(c, b)
  for b in range(NBUF):
    write_wait(b)


@jax.jit
def kernel(input_ids, token_embedding, position_embedding):
  ids_flat = input_ids.astype(jnp.int32).reshape(TOTAL)

  mesh = plsc.VectorSubcoreMesh(core_axis_name="c", subcore_axis_name="s")
  f = pl.kernel(
      _emb_body,
      out_type=jax.ShapeDtypeStruct((TOTAL, EMBED), jnp.float32),
      mesh=mesh,
      scratch_types=[
          pltpu.VMEM((ROWS_PER_WORKER,), jnp.int32),
          pltpu.VMEM((CHUNK, EMBED), jnp.float32),
          pltpu.VMEM((CHUNK, EMBED), jnp.float32),
          pltpu.VMEM((CHUNK, EMBED), jnp.float32),
          pltpu.VMEM((NUM_POS, EMBED), jnp.float32),
          pltpu.SemaphoreType.DMA,
          pltpu.SemaphoreType.DMA,
          pltpu.SemaphoreType.DMA,
          pltpu.SemaphoreType.DMA,
          pltpu.SemaphoreType.DMA,
          pltpu.SemaphoreType.DMA,
      ],
  )
  out = f(ids_flat, token_embedding, position_embedding)
  return out.reshape(BATCH, SEQ, EMBED)


# speedup vs baseline: 1.0525x; 1.0525x over previous
import functools

import jax
import jax.numpy as jnp
from jax import lax
from jax.experimental import pallas as pl
from jax.experimental.pallas import tpu as pltpu
from jax.experimental.pallas import tpu_sc as plsc

VOCAB = 49408
EMBED = 512
NUM_POS = 77
BATCH = 1024
SEQ = 77
TOTAL = BATCH * SEQ
NUM_WORKERS = 32
ROWS_PER_WORKER = TOTAL // NUM_WORKERS
CHUNK = 56
NCHUNKS = ROWS_PER_WORKER // CHUNK
NBUF = 3
SUB = 8
NSUB = CHUNK // SUB
LANES = 16
VREGS_PER_ROW = EMBED // LANES


def _emb_body(ids_hbm, table_hbm, pos_hbm, out_hbm, idx_v, b0, b1, b2, pos_v,
              g0, g1, g2, w0, w1, w2):
  bufs = (b0, b1, b2)
  gsems = (g0, g1, g2)
  wsems = (w0, w1, w2)
  num_cores = 2
  wid = lax.axis_index("s") * num_cores + lax.axis_index("c")
  base = wid * ROWS_PER_WORKER

  pltpu.sync_copy(ids_hbm.at[pl.ds(base, ROWS_PER_WORKER)], idx_v)
  pltpu.sync_copy(pos_hbm, pos_v)

  def gather_start(c, b):
    for i in range(NSUB):
      pltpu.async_copy(
          table_hbm.at[idx_v.at[pl.ds(c * CHUNK + i * SUB, SUB)]],
          bufs[b].at[pl.ds(i * SUB, SUB)], gsems[b])

  def process(c, b):
    s0 = lax.rem(c * CHUNK, NUM_POS)
    buf = bufs[b]
    for i in range(NSUB):
      pltpu.make_async_copy(
          table_hbm.at[idx_v.at[pl.ds(0, SUB)]],
          buf.at[pl.ds(i * SUB, SUB)], gsems[b]).wait()

      def body(k, cc):
        rk = s0 + k
        r = lax.select(rk >= NUM_POS, rk - NUM_POS, rk)
        for j in range(VREGS_PER_ROW):
          sl = pl.ds(j * LANES, LANES)
          buf[k, sl] = buf[k, sl] + pos_v[r, sl]
        return cc

      lax.fori_loop(i * SUB, (i + 1) * SUB, body, 0)

  def write_start(c, b):
    pltpu.async_copy(bufs[b], out_hbm.at[pl.ds(base + c * CHUNK, CHUNK)],
                     wsems[b])

  def write_wait(b):
    pltpu.make_async_copy(
        bufs[b], out_hbm.at[pl.ds(0, CHUNK)], wsems[b]).wait()

  gather_start(0, 0)
  gather_start(1, 1)

  def triple(t, carry):
    for b in range(NBUF):
      c = 3 * t + b

      @pl.when(c < NCHUNKS)
      def _step():
        process(c, b)
        write_start(c, b)

        @pl.when(c >= 1)
        def _free():
          write_wait((b + 2) % NBUF)

        @pl.when(c + 2 < NCHUNKS)
        def _next():
          gather_start(c + 2, (b + 2) % NBUF)

    return carry

  lax.fori_loop(0, (NCHUNKS + NBUF) // NBUF, triple, 0)

  write_wait((NCHUNKS - 1) % NBUF)


@jax.jit
def kernel(input_ids, token_embedding, position_embedding):
  ids_flat = input_ids.astype(jnp.int32).reshape(TOTAL)

  mesh = plsc.VectorSubcoreMesh(core_axis_name="c", subcore_axis_name="s")
  f = pl.kernel(
      _emb_body,
      out_type=jax.ShapeDtypeStruct((TOTAL, EMBED), jnp.float32),
      mesh=mesh,
      scratch_types=[
          pltpu.VMEM((ROWS_PER_WORKER,), jnp.int32),
          pltpu.VMEM((CHUNK, EMBED), jnp.float32),
          pltpu.VMEM((CHUNK, EMBED), jnp.float32),
          pltpu.VMEM((CHUNK, EMBED), jnp.float32),
          pltpu.VMEM((NUM_POS, EMBED), jnp.float32),
          pltpu.SemaphoreType.DMA,
          pltpu.SemaphoreType.DMA,
          pltpu.SemaphoreType.DMA,
          pltpu.SemaphoreType.DMA,
          pltpu.SemaphoreType.DMA,
          pltpu.SemaphoreType.DMA,
      ],
  )
  out = f(ids_flat, token_embedding, position_embedding)
  return out.reshape(BATCH, SEQ, EMBED)


# speedup vs baseline: 1.4423x; 1.3703x over previous
import functools

import jax
import jax.numpy as jnp
from jax import lax
from jax.experimental import pallas as pl
from jax.experimental.pallas import tpu as pltpu
from jax.experimental.pallas import tpu_sc as plsc

VOCAB = 49408
EMBED = 512
NUM_POS = 77
BATCH = 1024
SEQ = 77
TOTAL = BATCH * SEQ
NUM_WORKERS = 32
ROWS_PER_WORKER = TOTAL // NUM_WORKERS
CHUNK = 56
NCHUNKS = ROWS_PER_WORKER // CHUNK
NBUF = 3


def _gather_body(ids_hbm, table_hbm, out_hbm,
                 i0, i1, i2, b0, b1, b2,
                 s0, s1, s2, g0, g1, g2, w0, w1, w2):
  ibufs = (i0, i1, i2)
  isems = (s0, s1, s2)
  bufs = (b0, b1, b2)
  gsems = (g0, g1, g2)
  wsems = (w0, w1, w2)
  num_cores = 2
  wid = lax.axis_index("s") * num_cores + lax.axis_index("c")
  base = wid * ROWS_PER_WORKER

  def i_start(c, b):
    pltpu.async_copy(
        ids_hbm.at[pl.ds(base + c * CHUNK, CHUNK)], ibufs[b], isems[b])

  def i_wait(b):
    pltpu.make_async_copy(
        ids_hbm.at[pl.ds(0, CHUNK)], ibufs[b], isems[b]).wait()

  def g_start(b):
    pltpu.async_copy(table_hbm.at[ibufs[b]], bufs[b], gsems[b])

  def g_wait(b):
    pltpu.make_async_copy(table_hbm.at[ibufs[b]], bufs[b], gsems[b]).wait()

  def w_start(c, b):
    pltpu.async_copy(
        bufs[b], out_hbm.at[pl.ds(base + c * CHUNK, CHUNK)], wsems[b])

  def w_wait(b):
    pltpu.make_async_copy(
        bufs[b], out_hbm.at[pl.ds(0, CHUNK)], wsems[b]).wait()

  i_start(0, 0)
  i_start(1, 1)
  i_wait(0)
  g_start(0)
  i_wait(1)
  g_start(1)

  def triple(t, carry):
    for b in range(NBUF):
      c = 3 * t + b

      @pl.when(c < NCHUNKS)
      def _step():
        g_wait(b)
        w_start(c, b)

        @pl.when(c >= 1)
        def _free():
          w_wait((b + 2) % NBUF)

        @pl.when(c + 2 < NCHUNKS)
        def _next():
          b2 = (b + 2) % NBUF
          i_start(c + 2, b2)
          i_wait(b2)
          g_start(b2)

    return carry

  lax.fori_loop(0, (NCHUNKS + NBUF) // NBUF, triple, 0)
  w_wait((NCHUNKS - 1) % NBUF)


def _add_body(gathered_ref, poscyc_ref, out_ref):
  out_ref[...] = gathered_ref[...] + poscyc_ref[...]


@jax.jit
def kernel(input_ids, token_embedding, position_embedding):
  ids_flat = input_ids.astype(jnp.int32).reshape(TOTAL)

  mesh = plsc.VectorSubcoreMesh(core_axis_name="c", subcore_axis_name="s")
  gather = pl.kernel(
      _gather_body,
      out_type=jax.ShapeDtypeStruct((TOTAL, EMBED), jnp.float32),
      mesh=mesh,
      scratch_types=[
          pltpu.VMEM((CHUNK,), jnp.int32),
          pltpu.VMEM((CHUNK,), jnp.int32),
          pltpu.VMEM((CHUNK,), jnp.int32),
          pltpu.VMEM((CHUNK, EMBED), jnp.float32),
          pltpu.VMEM((CHUNK, EMBED), jnp.float32),
          pltpu.VMEM((CHUNK, EMBED), jnp.float32),
          pltpu.SemaphoreType.DMA,
          pltpu.SemaphoreType.DMA,
          pltpu.SemaphoreType.DMA,
          pltpu.SemaphoreType.DMA,
          pltpu.SemaphoreType.DMA,
          pltpu.SemaphoreType.DMA,
          pltpu.SemaphoreType.DMA,
          pltpu.SemaphoreType.DMA,
          pltpu.SemaphoreType.DMA,
      ],
  )
  gathered = gather(ids_flat, token_embedding)

  poscyc = jnp.tile(position_embedding, (ROWS_PER_WORKER // NUM_POS, 1))
  out = pl.pallas_call(
      _add_body,
      out_shape=jax.ShapeDtypeStruct((TOTAL, EMBED), jnp.float32),
      grid=(TOTAL // ROWS_PER_WORKER,),
      in_specs=[
          pl.BlockSpec((ROWS_PER_WORKER, EMBED), lambda i: (i, 0)),
          pl.BlockSpec((ROWS_PER_WORKER, EMBED), lambda i: (0, 0)),
      ],
      out_specs=pl.BlockSpec((ROWS_PER_WORKER, EMBED), lambda i: (i, 0)),
  )(gathered, poscyc)
  return out.reshape(BATCH, SEQ, EMBED)
